# baseline (speedup 1.0000x reference)
import numpy as np
import jax
import jax.numpy as jnp
from jax.experimental import pallas as pl


def _pool_matrix_pair(H, W, OH, OW):
    S = H * W
    bh, bw = H // OH, W // OW
    P = np.zeros((2 * S, 2 * OH * OW), np.float32)
    for c in range(2):
        for h in range(H):
            for w in range(W):
                P[c * S + h * W + w,
                  c * OH * OW + (h // bh) * OW + (w // bw)] = 1.0
    return P


def _pool_body(x_ref, p_ref, o_ref):
    o_ref[...] = jnp.dot(x_ref[...], p_ref[...],
                         preferred_element_type=jnp.float32,
                         precision=jax.lax.Precision.HIGHEST)


def _head_body(f_ref, w1_ref, b1_ref, w2_ref, b2_ref, g_ref, i_ref):
    scale = jnp.float32(1.0 / 36.0)
    h = jax.lax.dot_general(f_ref[...] * scale, w1_ref[...],
                            (((1,), (1,)), ((), ())),
                            preferred_element_type=jnp.float32) + b1_ref[...]
    h = jnp.maximum(h, 0.0)
    logits = jax.lax.dot_general(h, w2_ref[...],
                                 (((1,), (1,)), ((), ())),
                                 preferred_element_type=jnp.float32) + b2_ref[...]
    B, E = logits.shape
    lane = jax.lax.broadcasted_iota(jnp.int32, (B, E), 1)
    m1 = jnp.max(logits, axis=-1, keepdims=True)
    i1 = jnp.min(jnp.where(logits == m1, lane, E), axis=-1, keepdims=True)
    masked = jnp.where(lane == i1, -jnp.inf, logits)
    m2 = jnp.max(masked, axis=-1, keepdims=True)
    i2 = jnp.min(jnp.where(masked == m2, lane, E), axis=-1, keepdims=True)
    e2 = jnp.exp(m2 - m1)
    g1 = 1.0 / (1.0 + e2)
    g2 = e2 / (1.0 + e2)
    g_ref[...] = (jnp.where(lane == i1, g1, 0.0)
                  + jnp.where(lane == i2, g2, 0.0))
    i_ref[...] = jnp.where(lane == 0, i1, 0) + jnp.where(lane == 1, i2, 0)


def kernel(x, W1, b1, W2, b2):
    B, C, H, W = x.shape
    E = W2.shape[0]
    OH = OW = 4
    S = H * W
    ROWS = B * C // 2
    x2 = x.reshape(ROWS, 2 * S)
    P = jnp.asarray(_pool_matrix_pair(H, W, OH, OW))
    BLK = 2048
    packed = pl.pallas_call(
        _pool_body,
        grid=(ROWS // BLK,),
        in_specs=[pl.BlockSpec((BLK, 2 * S), lambda i: (i, 0)),
                  pl.BlockSpec((2 * S, 2 * OH * OW), lambda i: (0, 0))],
        out_specs=pl.BlockSpec((BLK, 2 * OH * OW), lambda i: (i, 0)),
        out_shape=jax.ShapeDtypeStruct((ROWS, 2 * OH * OW), jnp.float32),
    )(x2, P)
    flat = packed.reshape(B, C * OH * OW)
    gates, ipad = pl.pallas_call(
        _head_body,
        out_shape=[jax.ShapeDtypeStruct((B, E), jnp.float32),
                   jax.ShapeDtypeStruct((B, E), jnp.int32)],
    )(flat, W1, b1.reshape(1, -1), W2, b2.reshape(1, -1))
    return gates, ipad[:, :2]


# speedup vs baseline: 1.0130x; 1.0130x over previous
import numpy as np
import jax
import jax.numpy as jnp
from jax.experimental import pallas as pl


def _pool_matrix(H, W, OH, OW, G):
    S = H * W
    bh, bw = H // OH, W // OW
    P = np.zeros((G * S, G * OH * OW), np.float32)
    for c in range(G):
        for h in range(H):
            for w in range(W):
                P[c * S + h * W + w,
                  c * OH * OW + (h // bh) * OW + (w // bw)] = 1.0
    return P


def _pool_body(x_ref, p_ref, o_ref):
    o_ref[...] = jnp.dot(x_ref[...], p_ref[...],
                         preferred_element_type=jnp.float32,
                         precision=jax.lax.Precision.HIGHEST)


def _head_body(f_ref, w1_ref, b1_ref, w2_ref, b2_ref, g_ref, i_ref):
    scale = jnp.float32(1.0 / 36.0)
    h = jax.lax.dot_general(f_ref[...] * scale, w1_ref[...],
                            (((1,), (1,)), ((), ())),
                            preferred_element_type=jnp.float32) + b1_ref[...]
    h = jnp.maximum(h, 0.0)
    logits = jax.lax.dot_general(h, w2_ref[...],
                                 (((1,), (1,)), ((), ())),
                                 preferred_element_type=jnp.float32) + b2_ref[...]
    B, E = logits.shape
    lane = jax.lax.broadcasted_iota(jnp.int32, (B, E), 1)
    m1 = jnp.max(logits, axis=-1, keepdims=True)
    i1 = jnp.min(jnp.where(logits == m1, lane, E), axis=-1, keepdims=True)
    masked = jnp.where(lane == i1, -jnp.inf, logits)
    m2 = jnp.max(masked, axis=-1, keepdims=True)
    i2 = jnp.min(jnp.where(masked == m2, lane, E), axis=-1, keepdims=True)
    e2 = jnp.exp(m2 - m1)
    g1 = 1.0 / (1.0 + e2)
    g2 = e2 / (1.0 + e2)
    g_ref[...] = (jnp.where(lane == i1, g1, 0.0)
                  + jnp.where(lane == i2, g2, 0.0))
    i_ref[...] = jnp.where(lane == 0, i1, 0) + jnp.where(lane == 1, i2, 0)


def kernel(x, W1, b1, W2, b2):
    B, C, H, W = x.shape
    E = W2.shape[0]
    OH = OW = 4
    G = 8
    S = H * W
    ROWS = B * C // G
    x2 = x.reshape(ROWS, G * S)
    P = jnp.asarray(_pool_matrix(H, W, OH, OW, G))
    BLK = 512
    packed = pl.pallas_call(
        _pool_body,
        grid=(ROWS // BLK,),
        in_specs=[pl.BlockSpec((BLK, G * S), lambda i: (i, 0)),
                  pl.BlockSpec((G * S, G * OH * OW), lambda i: (0, 0))],
        out_specs=pl.BlockSpec((BLK, G * OH * OW), lambda i: (i, 0)),
        out_shape=jax.ShapeDtypeStruct((ROWS, G * OH * OW), jnp.float32),
    )(x2, P)
    flat = packed.reshape(B, C * OH * OW)
    gates, ipad = pl.pallas_call(
        _head_body,
        out_shape=[jax.ShapeDtypeStruct((B, E), jnp.float32),
                   jax.ShapeDtypeStruct((B, E), jnp.int32)],
    )(flat, W1, b1.reshape(1, -1), W2, b2.reshape(1, -1))
    return gates, ipad[:, :2]


# speedup vs baseline: 1.1340x; 1.1195x over previous
import jax
import jax.numpy as jnp
from jax.experimental import pallas as pl


def _pool_body(x_ref, o_ref):
    x = x_ref[...]
    cells = []
    for ph in range(4):
        band = jnp.sum(x[:, 6 * ph:6 * ph + 6, :], axis=1)
        for pw in range(4):
            cells.append(jnp.sum(band[:, 6 * pw:6 * pw + 6],
                                 axis=1, keepdims=True))
    o_ref[...] = jnp.concatenate(cells, axis=1)


def _head_body(f_ref, w1_ref, b1_ref, w2_ref, b2_ref, g_ref, i_ref):
    scale = jnp.float32(1.0 / 36.0)
    h = jax.lax.dot_general(f_ref[...] * scale, w1_ref[...],
                            (((1,), (1,)), ((), ())),
                            preferred_element_type=jnp.float32) + b1_ref[...]
    h = jnp.maximum(h, 0.0)
    logits = jax.lax.dot_general(h, w2_ref[...],
                                 (((1,), (1,)), ((), ())),
                                 preferred_element_type=jnp.float32) + b2_ref[...]
    B, E = logits.shape
    lane = jax.lax.broadcasted_iota(jnp.int32, (B, E), 1)
    m1 = jnp.max(logits, axis=-1, keepdims=True)
    i1 = jnp.min(jnp.where(logits == m1, lane, E), axis=-1, keepdims=True)
    masked = jnp.where(lane == i1, -jnp.inf, logits)
    m2 = jnp.max(masked, axis=-1, keepdims=True)
    i2 = jnp.min(jnp.where(masked == m2, lane, E), axis=-1, keepdims=True)
    e2 = jnp.exp(m2 - m1)
    g1 = 1.0 / (1.0 + e2)
    g2 = e2 / (1.0 + e2)
    g_ref[...] = (jnp.where(lane == i1, g1, 0.0)
                  + jnp.where(lane == i2, g2, 0.0))
    i_ref[...] = jnp.where(lane == 0, i1, 0) + jnp.where(lane == 1, i2, 0)


def kernel(x, W1, b1, W2, b2):
    B, C, H, W = x.shape
    E = W2.shape[0]
    OH = OW = 4
    ROWS = B * C
    xr = x.reshape(ROWS, H, W)
    BLK = 1024
    pooled = pl.pallas_call(
        _pool_body,
        grid=(ROWS // BLK,),
        in_specs=[pl.BlockSpec((BLK, H, W), lambda i: (i, 0, 0))],
        out_specs=pl.BlockSpec((BLK, OH * OW), lambda i: (i, 0)),
        out_shape=jax.ShapeDtypeStruct((ROWS, OH * OW), jnp.float32),
    )(xr)
    flat = pooled.reshape(B, C * OH * OW)
    gates, ipad = pl.pallas_call(
        _head_body,
        out_shape=[jax.ShapeDtypeStruct((B, E), jnp.float32),
                   jax.ShapeDtypeStruct((B, E), jnp.int32)],
    )(flat, W1, b1.reshape(1, -1), W2, b2.reshape(1, -1))
    return gates, ipad[:, :2]
